# baseline (speedup 1.0000x reference)
import jax
import jax.numpy as jnp
from jax import lax
from jax.experimental import pallas as pl
from jax.experimental.pallas import tpu as pltpu
from jax.experimental.pallas import tpu_sc as plsc

N_POS = 8192
H = 1024
B = 4
S = 8192
B_TOTAL = B * S
NC = 2
NS = 16
NW = NC * NS
B_PER_W = B_TOTAL // NW
CHUNK = 8
N_CHUNKS = B_PER_W // CHUNK
NBUF = 8
N_GROUPS = N_CHUNKS // NBUF
LANES = 16
VPR = H // LANES
VPC = CHUNK * VPR


def _gather_body(x_hbm, table_hbm, scal_hbm, out_hbm, idx_v, scal_v,
                 *bufs_and_sems):
    bufs = bufs_and_sems[:NBUF]
    gsems = bufs_and_sems[NBUF:2 * NBUF]
    osems = bufs_and_sems[2 * NBUF:3 * NBUF]
    wid = lax.axis_index("s") * NC + lax.axis_index("c")
    base = wid * B_PER_W
    pltpu.sync_copy(x_hbm.at[wid], idx_v)
    pltpu.sync_copy(scal_hbm, scal_v.at[pl.ds(0, 1)])
    s = scal_v[...][0]

    def gather_start(ci, k):
        pltpu.async_copy(table_hbm.at[idx_v.at[ci]], bufs[k], gsems[k])

    def gather_wait(ci, k):
        pltpu.make_async_copy(table_hbm.at[idx_v.at[ci]], bufs[k], gsems[k]).wait()

    def out_start(ci, k):
        pltpu.async_copy(bufs[k], out_hbm.at[pl.ds(base + ci * CHUNK, CHUNK)], osems[k])

    def out_wait(ci, k):
        pltpu.make_async_copy(
            bufs[k], out_hbm.at[pl.ds(base + ci * CHUNK, CHUNK)], osems[k]
        ).wait()

    def scale_buf(k):
        buf = bufs[k]

        @plsc.parallel_loop(0, VPC, 1, unroll=16)
        def _(i):
            r = jnp.right_shift(i, 6)
            off = jnp.bitwise_and(i, VPR - 1) * LANES
            sl = pl.ds(off, LANES)
            buf[r, sl] = buf[r, sl] * s

    for k in range(NBUF - 1):
        gather_start(k, k)

    def step(g, carry):
        for k in range(NBUF):
            ci = g * NBUF + k
            b = k
            br = (k + NBUF - 1) % NBUF
            gather_wait(ci, b)
            if k == 0:
                @pl.when(g > 0)
                def _():
                    out_wait(ci - 1, br)
            else:
                out_wait(ci - 1, br)
            if k == 0:
                gather_start(ci + NBUF - 1, br)
            else:
                @pl.when(g < N_GROUPS - 1)
                def _():
                    gather_start(ci + NBUF - 1, br)
            scale_buf(b)
            out_start(ci, b)
        return carry

    lax.fori_loop(0, N_GROUPS, step, 0)
    out_wait(N_CHUNKS - 1, (N_CHUNKS - 1) % NBUF)


@jax.jit
def _lookup(x3d, table3d, scaling):
    mesh = plsc.VectorSubcoreMesh(core_axis_name="c", subcore_axis_name="s")
    f = pl.kernel(
        _gather_body,
        out_type=jax.ShapeDtypeStruct((B_TOTAL, H), jnp.float32),
        mesh=mesh,
        scratch_types=(
            [pltpu.VMEM((N_CHUNKS, CHUNK), jnp.int32),
             pltpu.VMEM((LANES,), jnp.float32)]
            + [pltpu.VMEM((CHUNK, H), jnp.float32) for _ in range(NBUF)]
            + [pltpu.SemaphoreType.DMA for _ in range(2 * NBUF)]
        ),
    )
    return f(x3d, table3d, scaling)


def kernel(x, table, scaling):
    x3d = x.reshape(NW, N_CHUNKS, CHUNK)
    out = _lookup(x3d, table, scaling)
    return out.reshape(B, S, H)


# speedup vs baseline: 1.0054x; 1.0054x over previous
import jax
import jax.numpy as jnp
from jax import lax
from jax.experimental import pallas as pl
from jax.experimental.pallas import tpu as pltpu
from jax.experimental.pallas import tpu_sc as plsc

N_POS = 8192
H = 1024
B = 4
S = 8192
B_TOTAL = B * S
NC = 2
NS = 16
NW = NC * NS
B_PER_W = B_TOTAL // NW
CHUNK = 16
N_CHUNKS = B_PER_W // CHUNK
NBUF = 4
N_GROUPS = N_CHUNKS // NBUF
LANES = 16
VPR = H // LANES
VPC = CHUNK * VPR


def _gather_body(x_hbm, table_hbm, scal_hbm, out_hbm, idx_v, scal_v,
                 *bufs_and_sems):
    bufs = bufs_and_sems[:NBUF]
    gsems = bufs_and_sems[NBUF:2 * NBUF]
    osems = bufs_and_sems[2 * NBUF:3 * NBUF]
    wid = lax.axis_index("s") * NC + lax.axis_index("c")
    base = wid * B_PER_W
    pltpu.sync_copy(x_hbm.at[wid], idx_v)
    pltpu.sync_copy(scal_hbm, scal_v.at[pl.ds(0, 1)])
    s = scal_v[...][0]

    def gather_start(ci, k):
        pltpu.async_copy(table_hbm.at[idx_v.at[ci]], bufs[k], gsems[k])

    def gather_wait(ci, k):
        pltpu.make_async_copy(table_hbm.at[idx_v.at[ci]], bufs[k], gsems[k]).wait()

    def out_start(ci, k):
        pltpu.async_copy(bufs[k], out_hbm.at[pl.ds(base + ci * CHUNK, CHUNK)], osems[k])

    def out_wait(ci, k):
        pltpu.make_async_copy(
            bufs[k], out_hbm.at[pl.ds(base + ci * CHUNK, CHUNK)], osems[k]
        ).wait()

    def scale_buf(k):
        buf = bufs[k]

        @plsc.parallel_loop(0, VPC, 1, unroll=8)
        def _(i):
            r = jnp.right_shift(i, 6)
            off = jnp.bitwise_and(i, VPR - 1) * LANES
            sl = pl.ds(off, LANES)
            buf[r, sl] = buf[r, sl] * s

    for k in range(NBUF - 1):
        gather_start(k, k)

    def step(g, carry):
        for k in range(NBUF):
            ci = g * NBUF + k
            b = k
            br = (k + NBUF - 1) % NBUF
            gather_wait(ci, b)
            if k == 0:
                @pl.when(g > 0)
                def _():
                    out_wait(ci - 1, br)
            else:
                out_wait(ci - 1, br)
            if k == 0:
                gather_start(ci + NBUF - 1, br)
            else:
                @pl.when(g < N_GROUPS - 1)
                def _():
                    gather_start(ci + NBUF - 1, br)
            scale_buf(b)
            out_start(ci, b)
        return carry

    lax.fori_loop(0, N_GROUPS, step, 0)
    out_wait(N_CHUNKS - 1, (N_CHUNKS - 1) % NBUF)


@jax.jit
def _lookup(x3d, table3d, scaling):
    mesh = plsc.VectorSubcoreMesh(core_axis_name="c", subcore_axis_name="s")
    f = pl.kernel(
        _gather_body,
        out_type=jax.ShapeDtypeStruct((B_TOTAL, H), jnp.float32),
        mesh=mesh,
        scratch_types=(
            [pltpu.VMEM((N_CHUNKS, CHUNK), jnp.int32),
             pltpu.VMEM((LANES,), jnp.float32)]
            + [pltpu.VMEM((CHUNK, H), jnp.float32) for _ in range(NBUF)]
            + [pltpu.SemaphoreType.DMA for _ in range(2 * NBUF)]
        ),
    )
    return f(x3d, table3d, scaling)


def kernel(x, table, scaling):
    x3d = x.reshape(NW, N_CHUNKS, CHUNK)
    out = _lookup(x3d, table, scaling)
    return out.reshape(B, S, H)
